# Initial kernel scaffold; baseline (speedup 1.0000x reference)
import jax
import jax.numpy as jnp
from jax.experimental import pallas as pl


def kernel(data, W_enc, b_enc, codebook, W_dec, b_dec):
    raise NotImplementedError("write your pallas kernel here")


# speedup vs baseline: 1.1793x; 1.1793x over previous
import functools

import jax
import jax.numpy as jnp
from jax import lax
from jax.experimental import pallas as pl
from jax.experimental.pallas import tpu as pltpu
from jax.experimental.pallas import tpu_sc as plsc

M = 32768
K = 384
C = 16384
DOWN = 4
MT = 512
CT = 4096
DPAD = 16

_bf = jnp.bfloat16
_f32 = jnp.float32


def _argmin_body(p_ref, we_ref, be_ref, cb_ref, idx_ref,
                 zl_ref, zn_ref, mn_ref, ix_ref):
    c = pl.program_id(1)

    @pl.when(c == 0)
    def _init():
        pre = lax.dot_general(p_ref[...].astype(_bf), we_ref[...].astype(_bf),
                              (((1,), (0,)), ((), ())),
                              preferred_element_type=_f32)
        z = jnp.tanh(pre + be_ref[...])
        zn_ref[...] = jnp.sum(z * z, axis=1, keepdims=True)
        zl_ref[...] = (2.0 * z).astype(_bf)
        mn_ref[...] = jnp.full((MT, 1), _f32(jnp.inf)).astype(_bf)
        ix_ref[...] = jnp.zeros((MT, 1), jnp.int32)

    cb = cb_ref[...]
    cn = jnp.sum(cb * cb, axis=1)
    s = lax.dot_general(zl_ref[...], cb, (((1,), (1,)), ((), ())),
                        preferred_element_type=_f32)
    dm = (zn_ref[...] - s) + cn[None, :]
    lmin = jnp.min(dm, axis=1, keepdims=True)
    larg = jnp.argmin(dm, axis=1).astype(jnp.int32)[:, None] + c * CT
    better = lmin < mn_ref[...].astype(_f32)
    ix_ref[...] = jnp.where(better, larg, ix_ref[...])
    mn_ref[...] = jnp.where(better, lmin.astype(_bf), mn_ref[...])

    @pl.when(c == pl.num_programs(1) - 1)
    def _emit():
        idx_ref[...] = ix_ref[...]


def _vq_argmin(patches, W_enc, b_enc, codebook):
    idx2 = pl.pallas_call(
        _argmin_body,
        grid=(M // MT, C // CT),
        in_specs=[
            pl.BlockSpec((MT, DOWN), lambda m, c: (m, 0)),
            pl.BlockSpec((DOWN, K), lambda m, c: (0, 0)),
            pl.BlockSpec((1, K), lambda m, c: (0, 0)),
            pl.BlockSpec((CT, K), lambda m, c: (c, 0)),
        ],
        out_specs=pl.BlockSpec((MT, 1), lambda m, c: (m, 0)),
        out_shape=jax.ShapeDtypeStruct((M, 1), jnp.int32),
        scratch_shapes=[
            pltpu.VMEM((MT, K), _bf),
            pltpu.VMEM((MT, 1), _f32),
            pltpu.VMEM((MT, 1), _bf),
            pltpu.VMEM((MT, 1), jnp.int32),
        ],
        compiler_params=pltpu.CompilerParams(
            dimension_semantics=("arbitrary", "arbitrary")),
    )(patches, W_enc, b_enc, codebook)
    return idx2[:, 0]


def _dec_body(cb_ref, wd_ref, bd_ref, out_ref):
    out_ref[...] = lax.dot_general(
        cb_ref[...], wd_ref[...], (((1,), (0,)), ((), ())),
        precision=lax.Precision.HIGHEST,
        preferred_element_type=_f32) + bd_ref[...]


def _decoded_codebook(codebook, W_dec_pad, b_dec_pad):
    return pl.pallas_call(
        _dec_body,
        grid=(C // CT,),
        in_specs=[
            pl.BlockSpec((CT, K), lambda i: (i, 0)),
            pl.BlockSpec((K, DPAD), lambda i: (0, 0)),
            pl.BlockSpec((1, DPAD), lambda i: (0, 0)),
        ],
        out_specs=pl.BlockSpec((CT, DPAD), lambda i: (i, 0)),
        out_shape=jax.ShapeDtypeStruct((C, DPAD), jnp.float32),
    )(codebook, W_dec_pad, b_dec_pad)


def _sc_gather(table, idx):
    info = plsc.get_sparse_core_info()
    nw = info.num_cores * info.num_subcores
    b_per_w = M // nw
    mesh = plsc.VectorSubcoreMesh(core_axis_name="c", subcore_axis_name="s")

    @functools.partial(
        pl.kernel, mesh=mesh,
        compiler_params=pltpu.CompilerParams(use_tc_tiling_on_sc=False),
        out_type=jax.ShapeDtypeStruct((M, DPAD), jnp.float32),
        scratch_types=[
            pltpu.VMEM((b_per_w,), jnp.int32),
            pltpu.VMEM((b_per_w, DPAD), jnp.float32),
            pltpu.SemaphoreType.DMA,
        ],
    )
    def k(table_hbm, idx_hbm, out_hbm, idx_v, rows_v, sem):
        wid = lax.axis_index("s") * info.num_cores + lax.axis_index("c")
        base = wid * b_per_w
        pltpu.sync_copy(idx_hbm.at[pl.ds(base, b_per_w)], idx_v)
        pltpu.async_copy(table_hbm.at[idx_v], rows_v, sem).wait()
        pltpu.sync_copy(rows_v, out_hbm.at[pl.ds(base, b_per_w)])

    return k(table, idx)


def kernel(data, W_enc, b_enc, codebook, W_dec, b_dec):
    patches = data.reshape(M, DOWN)
    idx = _vq_argmin(patches, W_enc, b_enc.reshape(1, K), codebook)
    W_dec_pad = jnp.pad(W_dec, ((0, 0), (0, DPAD - DOWN)))
    b_dec_pad = jnp.pad(b_dec, (0, DPAD - DOWN)).reshape(1, DPAD)
    dec_cb = _decoded_codebook(codebook, W_dec_pad, b_dec_pad)
    rows = _sc_gather(dec_cb, idx)
    recon = rows[:, :DOWN].reshape(-1)
    return recon, idx


# speedup vs baseline: 1.4034x; 1.1900x over previous
import functools

import jax
import jax.numpy as jnp
from jax import lax
from jax.experimental import pallas as pl
from jax.experimental.pallas import tpu as pltpu
from jax.experimental.pallas import tpu_sc as plsc

M = 32768
K = 384
C = 16384
DOWN = 4
MT = 512
CT = 4096
DPAD = 16

_bf = jnp.bfloat16
_f32 = jnp.float32


def _argmin_body(p_ref, we_ref, be_ref, cb_ref, cn_ref, idx_ref,
                 zl_ref, zn_ref, mn_ref, ix_ref):
    c = pl.program_id(1)

    @pl.when(c == 0)
    def _init():
        pre = lax.dot_general(p_ref[...].astype(_bf), we_ref[...].astype(_bf),
                              (((1,), (0,)), ((), ())),
                              preferred_element_type=_f32)
        z = jnp.tanh(pre + be_ref[...])
        zn_ref[...] = jnp.sum(z * z, axis=1, keepdims=True)
        zl_ref[...] = (2.0 * z).astype(_bf)
        mn_ref[...] = jnp.full((MT, 1), _f32(jnp.inf)).astype(_bf)
        ix_ref[...] = jnp.zeros((MT, 1), jnp.int32)

    s = lax.dot_general(zl_ref[...], cb_ref[...], (((1,), (1,)), ((), ())),
                        preferred_element_type=_f32)
    dm = (zn_ref[...] - s) + cn_ref[...]
    lmin = jnp.min(dm, axis=1, keepdims=True)
    larg = jnp.argmin(dm, axis=1).astype(jnp.int32)[:, None] + c * CT
    better = lmin < mn_ref[...].astype(_f32)
    ix_ref[...] = jnp.where(better, larg, ix_ref[...])
    mn_ref[...] = jnp.where(better, lmin.astype(_bf), mn_ref[...])

    @pl.when(c == pl.num_programs(1) - 1)
    def _emit():
        idx_ref[...] = ix_ref[...]


def _vq_argmin(patches, W_enc, b_enc, codebook_bf, cnorm):
    idx2 = pl.pallas_call(
        _argmin_body,
        grid=(M // MT, C // CT),
        in_specs=[
            pl.BlockSpec((MT, DOWN), lambda m, c: (m, 0)),
            pl.BlockSpec((DOWN, K), lambda m, c: (0, 0)),
            pl.BlockSpec((1, K), lambda m, c: (0, 0)),
            pl.BlockSpec((CT, K), lambda m, c: (c, 0)),
            pl.BlockSpec((1, CT), lambda m, c: (0, c)),
        ],
        out_specs=pl.BlockSpec((MT, 1), lambda m, c: (m, 0)),
        out_shape=jax.ShapeDtypeStruct((M, 1), jnp.int32),
        scratch_shapes=[
            pltpu.VMEM((MT, K), _bf),
            pltpu.VMEM((MT, 1), _f32),
            pltpu.VMEM((MT, 1), _bf),
            pltpu.VMEM((MT, 1), jnp.int32),
        ],
        compiler_params=pltpu.CompilerParams(
            dimension_semantics=("arbitrary", "arbitrary")),
    )(patches, W_enc, b_enc, codebook_bf, cnorm)
    return idx2[:, 0]


def _dec_body(cb_ref, wd_ref, bd_ref, out_ref, cn_ref):
    cb = cb_ref[...]
    out_ref[...] = lax.dot_general(
        cb, wd_ref[...], (((1,), (0,)), ((), ())),
        precision=lax.Precision.HIGHEST,
        preferred_element_type=_f32) + bd_ref[...]
    cn_ref[...] = jnp.sum(cb * cb, axis=1)[None, :]


def _decoded_codebook(codebook, W_dec_pad, b_dec_pad):
    return pl.pallas_call(
        _dec_body,
        grid=(C // CT,),
        in_specs=[
            pl.BlockSpec((CT, K), lambda i: (i, 0)),
            pl.BlockSpec((K, DPAD), lambda i: (0, 0)),
            pl.BlockSpec((1, DPAD), lambda i: (0, 0)),
        ],
        out_specs=[
            pl.BlockSpec((CT, DPAD), lambda i: (i, 0)),
            pl.BlockSpec((1, CT), lambda i: (0, i)),
        ],
        out_shape=[
            jax.ShapeDtypeStruct((C, DPAD), jnp.float32),
            jax.ShapeDtypeStruct((1, C), jnp.float32),
        ],
    )(codebook, W_dec_pad, b_dec_pad)


def _sc_gather(table, idx):
    info = plsc.get_sparse_core_info()
    nw = info.num_cores * info.num_subcores
    b_per_w = M // nw
    mesh = plsc.VectorSubcoreMesh(core_axis_name="c", subcore_axis_name="s")

    @functools.partial(
        pl.kernel, mesh=mesh,
        compiler_params=pltpu.CompilerParams(use_tc_tiling_on_sc=False),
        out_type=jax.ShapeDtypeStruct((M, DPAD), jnp.float32),
        scratch_types=[
            pltpu.VMEM((b_per_w,), jnp.int32),
            pltpu.VMEM((b_per_w, DPAD), jnp.float32),
            pltpu.SemaphoreType.DMA,
        ],
    )
    def k(table_hbm, idx_hbm, out_hbm, idx_v, rows_v, sem):
        wid = lax.axis_index("s") * info.num_cores + lax.axis_index("c")
        base = wid * b_per_w
        pltpu.sync_copy(idx_hbm.at[pl.ds(base, b_per_w)], idx_v)
        pltpu.async_copy(table_hbm.at[idx_v], rows_v, sem).wait()
        pltpu.sync_copy(rows_v, out_hbm.at[pl.ds(base, b_per_w)])

    return k(table, idx)


def kernel(data, W_enc, b_enc, codebook, W_dec, b_dec):
    patches = data.reshape(M, DOWN)
    W_dec_pad = jnp.pad(W_dec, ((0, 0), (0, DPAD - DOWN)))
    b_dec_pad = jnp.pad(b_dec, (0, DPAD - DOWN)).reshape(1, DPAD)
    dec_cb, cnorm = _decoded_codebook(codebook, W_dec_pad, b_dec_pad)
    idx = _vq_argmin(patches, W_enc, b_enc.reshape(1, K),
                     codebook.astype(_bf), cnorm)
    rows = _sc_gather(dec_cb, idx)
    recon = rows[:, :DOWN].reshape(-1)
    return recon, idx


# speedup vs baseline: 1.4183x; 1.0106x over previous
import functools

import jax
import jax.numpy as jnp
from jax import lax
from jax.experimental import pallas as pl
from jax.experimental.pallas import tpu as pltpu
from jax.experimental.pallas import tpu_sc as plsc

M = 32768
K = 384
C = 16384
DOWN = 4
MT = 1024
CT = 4096
DPAD = 16

_bf = jnp.bfloat16
_f32 = jnp.float32


def _argmin_body(p_ref, we_ref, be_ref, cb_ref, cn_ref, idx_ref,
                 zl_ref, zn_ref, mn_ref, ix_ref):
    c = pl.program_id(1)

    @pl.when(c == 0)
    def _init():
        pre = lax.dot_general(p_ref[...].astype(_bf), we_ref[...].astype(_bf),
                              (((1,), (0,)), ((), ())),
                              preferred_element_type=_f32)
        z = jnp.tanh(pre + be_ref[...])
        zn_ref[...] = jnp.sum(z * z, axis=1, keepdims=True)
        zl_ref[...] = (2.0 * z).astype(_bf)
        mn_ref[...] = jnp.full((MT, 1), _f32(jnp.inf)).astype(_bf)
        ix_ref[...] = jnp.zeros((MT, 1), jnp.int32)

    s = lax.dot_general(zl_ref[...], cb_ref[...], (((1,), (1,)), ((), ())),
                        preferred_element_type=_f32)
    dm = (zn_ref[...] - s) + cn_ref[...]
    lmin = jnp.min(dm, axis=1, keepdims=True)
    larg = jnp.argmin(dm, axis=1).astype(jnp.int32)[:, None] + c * CT
    better = lmin < mn_ref[...].astype(_f32)
    ix_ref[...] = jnp.where(better, larg, ix_ref[...])
    mn_ref[...] = jnp.where(better, lmin.astype(_bf), mn_ref[...])

    @pl.when(c == pl.num_programs(1) - 1)
    def _emit():
        idx_ref[...] = ix_ref[...]


def _vq_argmin(patches, W_enc, b_enc, codebook_bf, cnorm):
    idx2 = pl.pallas_call(
        _argmin_body,
        grid=(M // MT, C // CT),
        in_specs=[
            pl.BlockSpec((MT, DOWN), lambda m, c: (m, 0)),
            pl.BlockSpec((DOWN, K), lambda m, c: (0, 0)),
            pl.BlockSpec((1, K), lambda m, c: (0, 0)),
            pl.BlockSpec((CT, K), lambda m, c: (c, 0)),
            pl.BlockSpec((1, CT), lambda m, c: (0, c)),
        ],
        out_specs=pl.BlockSpec((MT, 1), lambda m, c: (m, 0)),
        out_shape=jax.ShapeDtypeStruct((M, 1), jnp.int32),
        scratch_shapes=[
            pltpu.VMEM((MT, K), _bf),
            pltpu.VMEM((MT, 1), _f32),
            pltpu.VMEM((MT, 1), _bf),
            pltpu.VMEM((MT, 1), jnp.int32),
        ],
        compiler_params=pltpu.CompilerParams(
            dimension_semantics=("arbitrary", "arbitrary")),
    )(patches, W_enc, b_enc, codebook_bf, cnorm)
    return idx2[:, 0]


def _dec_body(cb_ref, wd_ref, bd_ref, out_ref, cn_ref):
    cb = cb_ref[...]
    out_ref[...] = lax.dot_general(
        cb, wd_ref[...], (((1,), (0,)), ((), ())),
        precision=lax.Precision.HIGHEST,
        preferred_element_type=_f32) + bd_ref[...]
    cn_ref[...] = jnp.sum(cb * cb, axis=1)[None, :]


def _decoded_codebook(codebook, W_dec_pad, b_dec_pad):
    return pl.pallas_call(
        _dec_body,
        grid=(C // CT,),
        in_specs=[
            pl.BlockSpec((CT, K), lambda i: (i, 0)),
            pl.BlockSpec((K, DPAD), lambda i: (0, 0)),
            pl.BlockSpec((1, DPAD), lambda i: (0, 0)),
        ],
        out_specs=[
            pl.BlockSpec((CT, DPAD), lambda i: (i, 0)),
            pl.BlockSpec((1, CT), lambda i: (0, i)),
        ],
        out_shape=[
            jax.ShapeDtypeStruct((C, DPAD), jnp.float32),
            jax.ShapeDtypeStruct((1, C), jnp.float32),
        ],
    )(codebook, W_dec_pad, b_dec_pad)


def _sc_gather(table, idx):
    info = plsc.get_sparse_core_info()
    nw = info.num_cores * info.num_subcores
    b_per_w = M // nw
    mesh = plsc.VectorSubcoreMesh(core_axis_name="c", subcore_axis_name="s")

    @functools.partial(
        pl.kernel, mesh=mesh,
        compiler_params=pltpu.CompilerParams(use_tc_tiling_on_sc=False),
        out_type=jax.ShapeDtypeStruct((M, DPAD), jnp.float32),
        scratch_types=[
            pltpu.VMEM((b_per_w,), jnp.int32),
            pltpu.VMEM((b_per_w, DPAD), jnp.float32),
            pltpu.SemaphoreType.DMA,
        ],
    )
    def k(table_hbm, idx_hbm, out_hbm, idx_v, rows_v, sem):
        wid = lax.axis_index("s") * info.num_cores + lax.axis_index("c")
        base = wid * b_per_w
        pltpu.sync_copy(idx_hbm.at[pl.ds(base, b_per_w)], idx_v)
        pltpu.async_copy(table_hbm.at[idx_v], rows_v, sem).wait()
        pltpu.sync_copy(rows_v, out_hbm.at[pl.ds(base, b_per_w)])

    return k(table, idx)


def kernel(data, W_enc, b_enc, codebook, W_dec, b_dec):
    patches = data.reshape(M, DOWN)
    W_dec_pad = jnp.pad(W_dec, ((0, 0), (0, DPAD - DOWN)))
    b_dec_pad = jnp.pad(b_dec, (0, DPAD - DOWN)).reshape(1, DPAD)
    dec_cb, cnorm = _decoded_codebook(codebook, W_dec_pad, b_dec_pad)
    idx = _vq_argmin(patches, W_enc, b_enc.reshape(1, K),
                     codebook.astype(_bf), cnorm)
    rows = _sc_gather(dec_cb, idx)
    recon = rows[:, :DOWN].reshape(-1)
    return recon, idx
